# baseline (speedup 1.0000x reference)
import functools

import jax
import jax.numpy as jnp
from jax import lax
from jax.experimental import pallas as pl
from jax.experimental.pallas import tpu as pltpu
from jax.experimental.pallas import tpu_sc as plsc

N_AID = 1000000
N_FACTORS = 32
BATCH = 16384
NPACK = N_FACTORS // 2

TC_W = 65536
TC_ROWS = TC_W // 4
TC_GRID = (N_AID + TC_W - 1) // TC_W
LIN_ROWS = TC_GRID * TC_ROWS
W_SHIFT = TC_W.bit_length() - 1
R_SHIFT = TC_ROWS.bit_length() - 1
R_MASK = TC_ROWS - 1

NUM_CORES = 2
NUM_SUBCORES = 16
NUM_WORKERS = NUM_CORES * NUM_SUBCORES
B_PER_W = BATCH // NUM_WORKERS
LANES = 16
BLOCKS = B_PER_W // LANES

_HI_MASK = -65536
_RND = 0x8000


def _linearize_tc(tt_ref, out_ref):
    blk = tt_ref[...]
    stacked = jnp.concatenate(
        [blk[:, q * TC_ROWS:(q + 1) * TC_ROWS] for q in range(4)], axis=0)
    jj = lax.broadcasted_iota(jnp.int32, (128, 128), 0)
    cc = lax.broadcasted_iota(jnp.int32, (128, 128), 1)
    tgt = 32 * ((cc % 64) // 16) + 2 * (cc % 16) + cc // 64
    sel = (jj == tgt).astype(jnp.float32)
    t = lax.dot_general(stacked, sel, (((0,), (0,)), ((), ())),
                        preferred_element_type=jnp.float32)
    bt = lax.bitcast_convert_type(t, jnp.int32)
    hi = jnp.bitwise_and(bt[:, :64] + _RND, _HI_MASK)
    lo = lax.shift_right_logical(bt[:, 64:] + _RND, 16)
    out_ref[...] = jnp.bitwise_or(hi, lo)


def _lin_table(table_t):
    return pl.pallas_call(
        _linearize_tc,
        grid=(TC_GRID,),
        in_specs=[pl.BlockSpec((N_FACTORS, TC_W), lambda c: (0, c))],
        out_specs=pl.BlockSpec((TC_ROWS, 64), lambda c: (c, 0)),
        out_shape=jax.ShapeDtypeStruct((LIN_ROWS, 64), jnp.int32),
    )(table_t)


def _mf_kernel(lin_hbm, aid_x_hbm, aid_y_hbm, coef_x_hbm, coef_y_hbm,
               out_hbm,
               idx_x_v, idx_y_v, rows_x_v, rows_y_v, cx_v, cy_v, out_v,
               sem_x, sem_y):
    wid = lax.axis_index("s") * NUM_CORES + lax.axis_index("c")
    base = wid * B_PER_W
    chunk = pl.ds(base, B_PER_W)

    pltpu.sync_copy(aid_x_hbm.at[chunk], idx_x_v)
    pltpu.sync_copy(aid_y_hbm.at[chunk], idx_y_v)

    for c in range(BLOCKS):
        sl = pl.ds(c * LANES, LANES)
        ax = idx_x_v[sl]
        ay = idx_y_v[sl]
        lx = jnp.bitwise_or(
            lax.shift_left(lax.shift_right_logical(ax, W_SHIFT), R_SHIFT),
            jnp.bitwise_and(ax, R_MASK))
        ly = jnp.bitwise_or(
            lax.shift_left(lax.shift_right_logical(ay, W_SHIFT), R_SHIFT),
            jnp.bitwise_and(ay, R_MASK))
        idx_x_v[sl] = jnp.bitwise_or(
            lax.shift_left(lx, 2),
            jnp.bitwise_and(lax.shift_right_logical(ax, R_SHIFT), 3))
        idx_y_v[sl] = jnp.bitwise_or(
            lax.shift_left(ly, 2),
            jnp.bitwise_and(lax.shift_right_logical(ay, R_SHIFT), 3))

    cpx = pltpu.async_copy(lin_hbm.at[idx_x_v], rows_x_v, sem_x)
    cpy = pltpu.async_copy(lin_hbm.at[idx_y_v], rows_y_v, sem_y)
    pltpu.sync_copy(coef_x_hbm.at[chunk], cx_v)
    pltpu.sync_copy(coef_y_hbm.at[chunk], cy_v)
    cpx.wait()
    cpy.wait()

    lane_iota = lax.iota(jnp.int32, LANES)

    def block_body(b, _):
        rows = b * LANES + lane_iota
        sl = pl.ds(b * LANES, LANES)
        acc = jnp.zeros((LANES,), jnp.float32)
        for w in range(NPACK):
            col = jnp.full((LANES,), w, jnp.int32)
            px = plsc.load_gather(rows_x_v, [rows, col])
            py = plsc.load_gather(rows_y_v, [rows, col])
            xe = plsc.bitcast(jnp.bitwise_and(px, _HI_MASK), jnp.float32)
            ye = plsc.bitcast(jnp.bitwise_and(py, _HI_MASK), jnp.float32)
            xo = plsc.bitcast(lax.shift_left(px, 16), jnp.float32)
            yo = plsc.bitcast(lax.shift_left(py, 16), jnp.float32)
            acc = acc + xe * ye + xo * yo
        out_v[sl] = acc * cx_v[sl] * cy_v[sl]
        return _

    lax.fori_loop(0, BLOCKS, block_body, 0)

    pltpu.sync_copy(out_v, out_hbm.at[chunk])


@jax.jit
def kernel(aid_x, aid_y, coef_x, coef_y, aid_embeddings):
    lin = _lin_table(aid_embeddings.T).reshape(4 * LIN_ROWS, NPACK)
    mesh = plsc.VectorSubcoreMesh(
        core_axis_name="c", subcore_axis_name="s",
        num_cores=NUM_CORES, num_subcores=NUM_SUBCORES)
    run = functools.partial(
        pl.kernel,
        out_type=jax.ShapeDtypeStruct((BATCH,), jnp.float32),
        mesh=mesh,
        compiler_params=pltpu.CompilerParams(
            needs_layout_passes=False, use_tc_tiling_on_sc=False),
        scratch_types=[
            pltpu.VMEM((B_PER_W,), jnp.int32),
            pltpu.VMEM((B_PER_W,), jnp.int32),
            pltpu.VMEM((B_PER_W, NPACK), jnp.int32),
            pltpu.VMEM((B_PER_W, NPACK), jnp.int32),
            pltpu.VMEM((B_PER_W,), jnp.float32),
            pltpu.VMEM((B_PER_W,), jnp.float32),
            pltpu.VMEM((B_PER_W,), jnp.float32),
            pltpu.SemaphoreType.DMA,
            pltpu.SemaphoreType.DMA,
        ],
    )(_mf_kernel)
    return run(lin, aid_x.astype(jnp.int32), aid_y.astype(jnp.int32),
               coef_x, coef_y)


# speedup vs baseline: 2.2686x; 2.2686x over previous
import functools

import jax
import jax.numpy as jnp
from jax import lax
from jax.experimental import pallas as pl
from jax.experimental.pallas import tpu as pltpu
from jax.experimental.pallas import tpu_sc as plsc

N_AID = 1000000
N_FACTORS = 32
BATCH = 16384
NPACK = N_FACTORS // 2

TC_W = 65536
HR = TC_W // 8
TC_GRID = (N_AID + TC_W - 1) // TC_W
LIN_ROWS = TC_GRID * HR
W_SHIFT = TC_W.bit_length() - 1
H_SHIFT = HR.bit_length() - 1
H_MASK = HR - 1

NUM_CORES = 2
NUM_SUBCORES = 16
NUM_WORKERS = NUM_CORES * NUM_SUBCORES
B_PER_W = BATCH // NUM_WORKERS
LANES = 16
BLOCKS = B_PER_W // LANES

_HI_MASK = -65536
_RND = 0x8000


def _linearize_tc(tt_ref, out_ref):
    blk = tt_ref[...]
    stacked = jnp.concatenate(
        [blk[:, p * HR:(p + 1) * HR] for p in range(8)], axis=0)
    jj = lax.broadcasted_iota(jnp.int32, (256, 128), 0)
    cc = lax.broadcasted_iota(jnp.int32, (256, 128), 1)
    tgt = 32 * (cc // 16) + 2 * (cc % 16)
    dims = (((0,), (0,)), ((), ()))
    t_e = lax.dot_general(stacked, (jj == tgt).astype(jnp.float32), dims,
                          preferred_element_type=jnp.float32)
    t_o = lax.dot_general(stacked, (jj == tgt + 1).astype(jnp.float32), dims,
                          preferred_element_type=jnp.float32)
    be = lax.bitcast_convert_type(t_e, jnp.int32)
    bo = lax.bitcast_convert_type(t_o, jnp.int32)
    hi = jnp.bitwise_and(be + _RND, _HI_MASK)
    lo = lax.shift_right_logical(bo + _RND, 16)
    out_ref[...] = jnp.bitwise_or(hi, lo)


def _lin_table(table_t):
    return pl.pallas_call(
        _linearize_tc,
        grid=(TC_GRID,),
        in_specs=[pl.BlockSpec((N_FACTORS, TC_W), lambda c: (0, c))],
        out_specs=pl.BlockSpec((HR, 128), lambda c: (c, 0)),
        out_shape=jax.ShapeDtypeStruct((LIN_ROWS, 128), jnp.int32),
    )(table_t)


def _mf_kernel(lin_hbm, aid_x_hbm, aid_y_hbm, coef_x_hbm, coef_y_hbm,
               out_hbm,
               idx_x_v, idx_y_v, rows_x_v, rows_y_v, cx_v, cy_v, out_v,
               sem_x, sem_y):
    wid = lax.axis_index("s") * NUM_CORES + lax.axis_index("c")
    base = wid * B_PER_W
    chunk = pl.ds(base, B_PER_W)

    pltpu.sync_copy(aid_x_hbm.at[chunk], idx_x_v)
    pltpu.sync_copy(aid_y_hbm.at[chunk], idx_y_v)

    for c in range(BLOCKS):
        sl = pl.ds(c * LANES, LANES)
        ax = idx_x_v[sl]
        ay = idx_y_v[sl]
        lx = jnp.bitwise_or(
            lax.shift_left(lax.shift_right_logical(ax, W_SHIFT), H_SHIFT),
            jnp.bitwise_and(ax, H_MASK))
        ly = jnp.bitwise_or(
            lax.shift_left(lax.shift_right_logical(ay, W_SHIFT), H_SHIFT),
            jnp.bitwise_and(ay, H_MASK))
        idx_x_v[sl] = jnp.bitwise_or(
            lax.shift_left(lx, 3),
            jnp.bitwise_and(lax.shift_right_logical(ax, H_SHIFT), 7))
        idx_y_v[sl] = jnp.bitwise_or(
            lax.shift_left(ly, 3),
            jnp.bitwise_and(lax.shift_right_logical(ay, H_SHIFT), 7))

    cpx = pltpu.async_copy(lin_hbm.at[idx_x_v], rows_x_v, sem_x)
    cpy = pltpu.async_copy(lin_hbm.at[idx_y_v], rows_y_v, sem_y)
    pltpu.sync_copy(coef_x_hbm.at[chunk], cx_v)
    pltpu.sync_copy(coef_y_hbm.at[chunk], cy_v)
    cpx.wait()
    cpy.wait()

    lane_iota = lax.iota(jnp.int32, LANES)

    def block_body(b, _):
        rows = b * LANES + lane_iota
        sl = pl.ds(b * LANES, LANES)
        acc = jnp.zeros((LANES,), jnp.float32)
        for w in range(NPACK):
            col = jnp.full((LANES,), w, jnp.int32)
            px = plsc.load_gather(rows_x_v, [rows, col])
            py = plsc.load_gather(rows_y_v, [rows, col])
            xe = plsc.bitcast(jnp.bitwise_and(px, _HI_MASK), jnp.float32)
            ye = plsc.bitcast(jnp.bitwise_and(py, _HI_MASK), jnp.float32)
            xo = plsc.bitcast(lax.shift_left(px, 16), jnp.float32)
            yo = plsc.bitcast(lax.shift_left(py, 16), jnp.float32)
            acc = acc + xe * ye + xo * yo
        out_v[sl] = acc * cx_v[sl] * cy_v[sl]
        return _

    lax.fori_loop(0, BLOCKS, block_body, 0)

    pltpu.sync_copy(out_v, out_hbm.at[chunk])


@jax.jit
def kernel(aid_x, aid_y, coef_x, coef_y, aid_embeddings):
    lin = _lin_table(aid_embeddings.T).reshape(8 * LIN_ROWS, NPACK)
    mesh = plsc.VectorSubcoreMesh(
        core_axis_name="c", subcore_axis_name="s",
        num_cores=NUM_CORES, num_subcores=NUM_SUBCORES)
    run = functools.partial(
        pl.kernel,
        out_type=jax.ShapeDtypeStruct((BATCH,), jnp.float32),
        mesh=mesh,
        compiler_params=pltpu.CompilerParams(
            needs_layout_passes=False, use_tc_tiling_on_sc=False),
        scratch_types=[
            pltpu.VMEM((B_PER_W,), jnp.int32),
            pltpu.VMEM((B_PER_W,), jnp.int32),
            pltpu.VMEM((B_PER_W, NPACK), jnp.int32),
            pltpu.VMEM((B_PER_W, NPACK), jnp.int32),
            pltpu.VMEM((B_PER_W,), jnp.float32),
            pltpu.VMEM((B_PER_W,), jnp.float32),
            pltpu.VMEM((B_PER_W,), jnp.float32),
            pltpu.SemaphoreType.DMA,
            pltpu.SemaphoreType.DMA,
        ],
    )(_mf_kernel)
    return run(lin, aid_x.astype(jnp.int32), aid_y.astype(jnp.int32),
               coef_x, coef_y)


# speedup vs baseline: 2.3473x; 1.0347x over previous
import functools

import jax
import jax.numpy as jnp
from jax import lax
from jax.experimental import pallas as pl
from jax.experimental.pallas import tpu as pltpu
from jax.experimental.pallas import tpu_sc as plsc

N_AID = 1000000
N_FACTORS = 32
BATCH = 16384
NPACK = N_FACTORS // 2

TC_W = 131072
HR = TC_W // 8
TC_GRID = (N_AID + TC_W - 1) // TC_W
LIN_ROWS = TC_GRID * HR
W_SHIFT = TC_W.bit_length() - 1
H_SHIFT = HR.bit_length() - 1
H_MASK = HR - 1

NUM_CORES = 2
NUM_SUBCORES = 16
NUM_WORKERS = NUM_CORES * NUM_SUBCORES
B_PER_W = BATCH // NUM_WORKERS
LANES = 16
BLOCKS = B_PER_W // LANES

_HI_MASK = -65536
_RND = 0x8000


def _linearize_tc(tt_ref, out_ref):
    blk = tt_ref[...]
    stacked = jnp.concatenate(
        [blk[:, p * HR:(p + 1) * HR] for p in range(8)], axis=0)
    jj = lax.broadcasted_iota(jnp.int32, (256, 128), 0)
    cc = lax.broadcasted_iota(jnp.int32, (256, 128), 1)
    tgt = 32 * (cc // 16) + 2 * (cc % 16)
    dims = (((0,), (0,)), ((), ()))
    t_e = lax.dot_general(stacked, (jj == tgt).astype(jnp.float32), dims,
                          preferred_element_type=jnp.float32)
    t_o = lax.dot_general(stacked, (jj == tgt + 1).astype(jnp.float32), dims,
                          preferred_element_type=jnp.float32)
    be = lax.bitcast_convert_type(t_e, jnp.int32)
    bo = lax.bitcast_convert_type(t_o, jnp.int32)
    hi = jnp.bitwise_and(be + _RND, _HI_MASK)
    lo = lax.shift_right_logical(bo + _RND, 16)
    out_ref[...] = jnp.bitwise_or(hi, lo)


def _lin_table(table_t):
    return pl.pallas_call(
        _linearize_tc,
        grid=(TC_GRID,),
        in_specs=[pl.BlockSpec((N_FACTORS, TC_W), lambda c: (0, c))],
        out_specs=pl.BlockSpec((HR, 128), lambda c: (c, 0)),
        out_shape=jax.ShapeDtypeStruct((LIN_ROWS, 128), jnp.int32),
    )(table_t)


def _mf_kernel(lin_hbm, aid_x_hbm, aid_y_hbm, coef_x_hbm, coef_y_hbm,
               out_hbm,
               idx_x_v, idx_y_v, rows_x_v, rows_y_v, cx_v, cy_v, out_v,
               sem_x, sem_y):
    wid = lax.axis_index("s") * NUM_CORES + lax.axis_index("c")
    base = wid * B_PER_W
    chunk = pl.ds(base, B_PER_W)

    pltpu.sync_copy(aid_x_hbm.at[chunk], idx_x_v)
    pltpu.sync_copy(aid_y_hbm.at[chunk], idx_y_v)

    for c in range(BLOCKS):
        sl = pl.ds(c * LANES, LANES)
        ax = idx_x_v[sl]
        ay = idx_y_v[sl]
        lx = jnp.bitwise_or(
            lax.shift_left(lax.shift_right_logical(ax, W_SHIFT), H_SHIFT),
            jnp.bitwise_and(ax, H_MASK))
        ly = jnp.bitwise_or(
            lax.shift_left(lax.shift_right_logical(ay, W_SHIFT), H_SHIFT),
            jnp.bitwise_and(ay, H_MASK))
        idx_x_v[sl] = jnp.bitwise_or(
            lax.shift_left(lx, 3),
            jnp.bitwise_and(lax.shift_right_logical(ax, H_SHIFT), 7))
        idx_y_v[sl] = jnp.bitwise_or(
            lax.shift_left(ly, 3),
            jnp.bitwise_and(lax.shift_right_logical(ay, H_SHIFT), 7))

    cpx = pltpu.async_copy(lin_hbm.at[idx_x_v], rows_x_v, sem_x)
    cpy = pltpu.async_copy(lin_hbm.at[idx_y_v], rows_y_v, sem_y)
    pltpu.sync_copy(coef_x_hbm.at[chunk], cx_v)
    pltpu.sync_copy(coef_y_hbm.at[chunk], cy_v)
    cpx.wait()
    cpy.wait()

    lane_iota = lax.iota(jnp.int32, LANES)

    def block_body(b, _):
        rows = b * LANES + lane_iota
        sl = pl.ds(b * LANES, LANES)
        acc = jnp.zeros((LANES,), jnp.float32)
        for w in range(NPACK):
            col = jnp.full((LANES,), w, jnp.int32)
            px = plsc.load_gather(rows_x_v, [rows, col])
            py = plsc.load_gather(rows_y_v, [rows, col])
            xe = plsc.bitcast(jnp.bitwise_and(px, _HI_MASK), jnp.float32)
            ye = plsc.bitcast(jnp.bitwise_and(py, _HI_MASK), jnp.float32)
            xo = plsc.bitcast(lax.shift_left(px, 16), jnp.float32)
            yo = plsc.bitcast(lax.shift_left(py, 16), jnp.float32)
            acc = acc + xe * ye + xo * yo
        out_v[sl] = acc * cx_v[sl] * cy_v[sl]
        return _

    lax.fori_loop(0, BLOCKS, block_body, 0)

    pltpu.sync_copy(out_v, out_hbm.at[chunk])


@jax.jit
def kernel(aid_x, aid_y, coef_x, coef_y, aid_embeddings):
    lin = _lin_table(aid_embeddings.T).reshape(8 * LIN_ROWS, NPACK)
    mesh = plsc.VectorSubcoreMesh(
        core_axis_name="c", subcore_axis_name="s",
        num_cores=NUM_CORES, num_subcores=NUM_SUBCORES)
    run = functools.partial(
        pl.kernel,
        out_type=jax.ShapeDtypeStruct((BATCH,), jnp.float32),
        mesh=mesh,
        compiler_params=pltpu.CompilerParams(
            needs_layout_passes=False, use_tc_tiling_on_sc=False),
        scratch_types=[
            pltpu.VMEM((B_PER_W,), jnp.int32),
            pltpu.VMEM((B_PER_W,), jnp.int32),
            pltpu.VMEM((B_PER_W, NPACK), jnp.int32),
            pltpu.VMEM((B_PER_W, NPACK), jnp.int32),
            pltpu.VMEM((B_PER_W,), jnp.float32),
            pltpu.VMEM((B_PER_W,), jnp.float32),
            pltpu.VMEM((B_PER_W,), jnp.float32),
            pltpu.SemaphoreType.DMA,
            pltpu.SemaphoreType.DMA,
        ],
    )(_mf_kernel)
    return run(lin, aid_x.astype(jnp.int32), aid_y.astype(jnp.int32),
               coef_x, coef_y)
